# baseline (speedup 1.0000x reference)
import functools

import jax
import jax.numpy as jnp
from jax import lax
from jax.experimental import pallas as pl
from jax.experimental.pallas import tpu as pltpu
from jax.experimental.pallas import tpu_sc as plsc

_B = 512
_NEG_INF = float("-inf")


def _sc_counts(idx):
    n = idx.shape[0]
    per = (n // 512) * 16
    nlast = n - 31 * per
    mesh = plsc.VectorSubcoreMesh(core_axis_name="c", subcore_axis_name="s")

    @functools.partial(
        pl.kernel, mesh=mesh,
        out_type=[jax.ShapeDtypeStruct((_B, 128), jnp.float32),
                  jax.ShapeDtypeStruct((_B, 128), jnp.float32)],
        scratch_types=[
            pltpu.VMEM((_B, 128), jnp.float32),
            pltpu.VMEM((nlast,), jnp.int32),
            pltpu.VMEM((_B // 128, 128), jnp.int32),
            pltpu.VMEM_SHARED((_B, 128), jnp.float32),
        ],
    )
    def k(idx_hbm, cnt_out0, cnt_out1, cnt, idxb, ident, shcnt):
        c = lax.axis_index("c")
        s = lax.axis_index("s")
        wid = c * 16 + s
        zero16 = jnp.zeros((16,), jnp.float32)
        one16 = jnp.ones((16,), jnp.float32)

        def zb(b, cz):
            for g in range(8):
                cnt[b, pl.ds(g * 16, 16)] = zero16
            return cz
        lax.fori_loop(0, _B, zb, 0)

        ii = lax.iota(jnp.int32, 16)
        for j in range(_B // 128):
            for t in range(8):
                ident[j, pl.ds(t * 16, 16)] = ii + (j * 128 + t * 16)

        @pl.when(s == 0)
        def _():
            pltpu.sync_copy(cnt, shcnt)
        plsc.subcore_barrier()

        pltpu.sync_copy(idx_hbm.at[pl.ds(wid * per, nlast)], idxb)

        def group(g16, cz):
            iv = idxb[pl.ds(g16 * 16, 16)]
            for j in range(16):
                plsc.addupdate(cnt.at[iv[j], pl.ds((j % 8) * 16, 16)], one16)
            return cz
        lax.fori_loop(0, per // 16, group, 0)

        @pl.when(wid == 31)
        def _():
            lax.fori_loop(per // 16, nlast // 16, group, 0)

        plsc.subcore_barrier()
        for j in range(_B // 128):
            pltpu.sync_copy(cnt.at[pl.ds(j * 128, 128)],
                            shcnt.at[ident.at[j]], add=True)
        plsc.subcore_barrier()

        @pl.when((s == 0) & (c == 0))
        def _():
            pltpu.sync_copy(shcnt, cnt_out0)

        @pl.when((s == 0) & (c != 0))
        def _():
            pltpu.sync_copy(shcnt, cnt_out1)

    return k(idx)


def _sig(x):
    return 1.0 / (1.0 + jnp.exp(-x))


def _pick_t(n, candidates=(1280, 640, 512, 500, 400, 320, 256, 128, 64, 32,
                           16, 8)):
    for t in candidates:
        if n % t == 0:
            return t
    return n


def _seg_range(idx):
    return jnp.min(idx), jnp.max(idx)


def _stats_body(f_ref, idx_ref, w1t_ref, b1_ref,
                hsum_ref, hsq_ref, ssum_ref, ssq_ref, cnt_ref,
                sacc_ref, qacc_ref, cacc_ref):
    i = pl.program_id(0)
    nsteps = pl.num_programs(0)

    @pl.when(i == 0)
    def _():
        hsum_ref[...] = jnp.zeros_like(hsum_ref)
        hsq_ref[...] = jnp.zeros_like(hsq_ref)
        sacc_ref[...] = jnp.zeros_like(sacc_ref)
        qacc_ref[...] = jnp.zeros_like(qacc_ref)
        cacc_ref[...] = jnp.zeros_like(cacc_ref)

    f = f_ref[...]
    h = jnp.dot(f, w1t_ref[...], preferred_element_type=jnp.float32) + b1_ref[...]
    hsum_ref[...] += jnp.sum(h, axis=0, keepdims=True)
    hsq_ref[...] += jnp.sum(h * h, axis=0, keepdims=True)

    idx = idx_ref[...]
    f2 = f * f
    smin, smax = _seg_range(idx)
    iota_w = lax.broadcasted_iota(jnp.int32, (1, _W), 1)

    def win(w, carry):
        base = smin + w * _W
        oh = ((idx - base) == iota_w).astype(jnp.float32)
        dng = lambda x: lax.dot_general(oh, x, (((0,), (0,)), ((), ())),
                                        preferred_element_type=jnp.float32)
        sacc_ref[pl.ds(base, _W), :] += dng(f)
        qacc_ref[pl.ds(base, _W), :] += dng(f2)
        cacc_ref[pl.ds(base, _W), :] += dng(jnp.ones_like(idx, jnp.float32))
        return carry

    lax.fori_loop(0, (smax - smin) // _W + 1, win, 0)

    @pl.when(i == nsteps - 1)
    def _():
        ssum_ref[...] = sacc_ref[:_B, :]
        ssq_ref[...] = qacc_ref[:_B, :]
        cnt_ref[...] = cacc_ref[:_B, :]


_W = 32


def _seg_softmax_window(oh, e, inw, lam, lneg, f, u2, base,
                        m_ref, den_ref, nums_ref, numf_ref, numu_ref):
    bmax = jnp.max(jnp.where(inw, e, _NEG_INF))
    safe_b = jnp.where(bmax == _NEG_INF, 0.0, bmax)
    exb = jnp.where(inw, jnp.exp(jnp.minimum(e - safe_b, 0.0)), 0.0)
    exn = exb * lneg
    dng = lambda x: lax.dot_general(oh, x, (((0,), (0,)), ((), ())),
                                    preferred_element_type=jnp.float32)
    den_add = dng(exb)
    nums_add = dng(exn)
    numf_add = dng((exb * lam) * f)
    numu_add = dng(exn * u2)
    present = den_add > 0.0
    m_old = m_ref[pl.ds(base, _W), :]
    m_new = jnp.where(present, jnp.maximum(m_old, safe_b), m_old)
    scale = jnp.where(present, jnp.exp(m_old - m_new), 1.0)
    corr = jnp.where(present, jnp.exp(safe_b - m_new), 0.0)
    den_ref[pl.ds(base, _W), :] = den_ref[pl.ds(base, _W), :] * scale + \
        den_add * corr
    nums_ref[pl.ds(base, _W), :] = nums_ref[pl.ds(base, _W), :] * scale + \
        nums_add * corr
    numf_ref[pl.ds(base, _W), :] = numf_ref[pl.ds(base, _W), :] * scale + \
        numf_add * corr
    numu_ref[pl.ds(base, _W), :] = numu_ref[pl.ds(base, _W), :] * scale + \
        numu_add * corr
    m_ref[pl.ds(base, _W), :] = m_new


def _s2s1_body(f_ref, u2_ref, idx_ref, noise_ref, a_ref, c_ref, w2_ref,
               b2_ref, mean_ref, std_ref,
               bih_ref, bhh_ref,
               lam_ref, pres_ref, r1_ref, s_out_ref, k_out_ref,
               q1_ref, m_ref, den_ref, nums_ref, numf_ref, numu_ref,
               sacc_ref, kacc_ref):
    i = pl.program_id(0)
    nsteps = pl.num_programs(0)
    t, d = f_ref.shape

    @pl.when(i == 0)
    def _():
        gates = bih_ref[...] + bhh_ref[...]
        ig = _sig(gates[:, 0:d])
        gg = jnp.tanh(gates[:, 2 * d:3 * d])
        og = _sig(gates[:, 3 * d:4 * d])
        cx1 = ig * gg
        q1_ref[...] = og * jnp.tanh(cx1)
        m_ref[...] = jnp.full_like(m_ref, _NEG_INF)
        den_ref[...] = jnp.zeros_like(den_ref)
        nums_ref[...] = jnp.zeros_like(nums_ref)
        numf_ref[...] = jnp.zeros_like(numf_ref)
        numu_ref[...] = jnp.zeros_like(numu_ref)
        sacc_ref[...] = jnp.zeros_like(sacc_ref)
        kacc_ref[...] = jnp.zeros_like(kacc_ref)
        pres_ref[...] = jnp.zeros_like(pres_ref)

    f = f_ref[...]
    u2 = u2_ref[...]
    hh = jnp.maximum(
        jnp.dot(f, a_ref[...], preferred_element_type=jnp.float32) + c_ref[...],
        0.0)
    p = jnp.sum(hh * w2_ref[...], axis=1, keepdims=True) + b2_ref[...]
    lam = _sig(noise_ref[...] + p)
    lam_ref[...] = lam
    pres_ref[...] += jnp.sum((p > 0.0).astype(jnp.float32), keepdims=True)
    lneg = 1.0 - lam
    lneg2 = lneg * lneg
    idx = idx_ref[...]
    q1 = q1_ref[...]
    dotfq = jnp.sum(f * q1, axis=1, keepdims=True)
    smin, smax = _seg_range(idx)
    iota_w = lax.broadcasted_iota(jnp.int32, (1, _W), 1)

    def win(w, carry):
        base = smin + w * _W
        loc = idx - base
        oh = (loc == iota_w).astype(jnp.float32)
        mu_n = jnp.dot(oh, mean_ref[pl.ds(base, _W), :],
                       preferred_element_type=jnp.float32)
        sg_n = jnp.dot(oh, std_ref[pl.ds(base, _W), :],
                       preferred_element_type=jnp.float32)
        muq = jnp.sum(mu_n * q1, axis=1, keepdims=True)
        u2dot = jnp.sum(u2 * (sg_n * q1), axis=1, keepdims=True)
        e = lam * dotfq + lneg * (muq + u2dot)
        inw = (loc >= 0) & (loc < _W)
        _seg_softmax_window(oh, e, inw, lam, lneg, f, u2, base,
                            m_ref, den_ref, nums_ref, numf_ref, numu_ref)
        dng = lambda x: lax.dot_general(oh, x, (((0,), (0,)), ((), ())),
                                        preferred_element_type=jnp.float32)
        sacc_ref[pl.ds(base, _W), :] += dng(lneg2)
        df = (lam * f + lneg * mu_n) - mu_n
        kacc_ref[pl.ds(base, _W), :] += dng(df * df)
        return carry

    lax.fori_loop(0, (smax - smin) // _W + 1, win, 0)

    @pl.when(i == nsteps - 1)
    def _():
        r1_ref[...] = (numf_ref[:_B, :] + nums_ref[:_B, :] * mean_ref[:_B, :] +
                       numu_ref[:_B, :] * std_ref[:_B, :]) / \
            (den_ref[:_B, :] + 1e-16)
        s_out_ref[...] = sacc_ref[:_B, :]
        k_out_ref[...] = kacc_ref[:_B, :]
        pres_ref[...] = pres_ref[...] * (1.0 / (nsteps * t))


def _s2s2_body(f_ref, u2_ref, idx_ref, lam_ref, mean_ref, std_ref,
               r1_ref, s_in_ref, k_in_ref,
               bih_ref, bhh_ref, wihlt_ref, wihrt_ref, whht_ref,
               w3lt_ref, w3rt_ref, b3_ref, w4t_ref, b4_ref, w5t_ref, b5_ref,
               preds_ref, kl_ref,
               q2_ref, m_ref, den_ref, nums_ref, numf_ref, numu_ref):
    i = pl.program_id(0)
    nsteps = pl.num_programs(0)
    d = f_ref.shape[1]

    @pl.when(i == 0)
    def _():
        gates0 = bih_ref[...] + bhh_ref[...]
        ig0 = _sig(gates0[:, 0:d])
        gg0 = jnp.tanh(gates0[:, 2 * d:3 * d])
        og0 = _sig(gates0[:, 3 * d:4 * d])
        cx1 = ig0 * gg0
        q1 = og0 * jnp.tanh(cx1)
        row = (jnp.dot(q1, wihlt_ref[...], preferred_element_type=jnp.float32)
               + jnp.dot(q1, whht_ref[...], preferred_element_type=jnp.float32)
               + bih_ref[...] + bhh_ref[...])
        gates = jnp.dot(r1_ref[...], wihrt_ref[...],
                        preferred_element_type=jnp.float32) + row
        ig = _sig(gates[:, 0:d])
        fg = _sig(gates[:, d:2 * d])
        gg = jnp.tanh(gates[:, 2 * d:3 * d])
        og = _sig(gates[:, 3 * d:4 * d])
        cx2 = fg * cx1 + ig * gg
        q2_ref[:_B, :] = og * jnp.tanh(cx2)
        q2_ref[_B:, :] = jnp.zeros_like(q2_ref[_B:, :])
        m_ref[...] = jnp.full_like(m_ref, _NEG_INF)
        den_ref[...] = jnp.zeros_like(den_ref)
        nums_ref[...] = jnp.zeros_like(nums_ref)
        numf_ref[...] = jnp.zeros_like(numf_ref)
        numu_ref[...] = jnp.zeros_like(numu_ref)

    f = f_ref[...]
    u2 = u2_ref[...]
    lam = lam_ref[...]
    lneg = 1.0 - lam
    idx = idx_ref[...]
    smin, smax = _seg_range(idx)
    iota_w = lax.broadcasted_iota(jnp.int32, (1, _W), 1)

    def win(w, carry):
        base = smin + w * _W
        loc = idx - base
        oh = (loc == iota_w).astype(jnp.float32)
        mu_n = jnp.dot(oh, mean_ref[pl.ds(base, _W), :],
                       preferred_element_type=jnp.float32)
        sg_n = jnp.dot(oh, std_ref[pl.ds(base, _W), :],
                       preferred_element_type=jnp.float32)
        q_n = jnp.dot(oh, q2_ref[pl.ds(base, _W), :],
                      preferred_element_type=jnp.float32)
        dotfq = jnp.sum(f * q_n, axis=1, keepdims=True)
        muq = jnp.sum(mu_n * q_n, axis=1, keepdims=True)
        u2dot = jnp.sum(u2 * (sg_n * q_n), axis=1, keepdims=True)
        e = lam * dotfq + lneg * (muq + u2dot)
        inw = (loc >= 0) & (loc < _W)
        _seg_softmax_window(oh, e, inw, lam, lneg, f, u2, base,
                            m_ref, den_ref, nums_ref, numf_ref, numu_ref)
        return carry

    lax.fori_loop(0, (smax - smin) // _W + 1, win, 0)

    @pl.when(i == nsteps - 1)
    def _():
        r2 = (numf_ref[:_B, :] + nums_ref[:_B, :] * mean_ref[:_B, :] +
              numu_ref[:_B, :] * std_ref[:_B, :]) / (den_ref[:_B, :] + 1e-16)
        q2 = q2_ref[:_B, :]
        x = jnp.maximum(
            jnp.dot(q2, w3lt_ref[...], preferred_element_type=jnp.float32) +
            jnp.dot(r2, w3rt_ref[...], preferred_element_type=jnp.float32) +
            b3_ref[...], 0.0)
        x = jnp.maximum(
            jnp.dot(x, w4t_ref[...], preferred_element_type=jnp.float32) +
            b4_ref[...], 0.0)
        preds_ref[...] = jnp.dot(x, w5t_ref[...],
                                 preferred_element_type=jnp.float32) + b5_ref[...]
        sg = std_ref[:_B, :]
        sge = (sg + 1e-07) * (sg + 1e-07)
        ms = jnp.mean((sg * sg) / sge, axis=1, keepdims=True)
        kl1 = 0.5 * s_in_ref[...] * ms
        kl2 = k_in_ref[...] / sge
        kl_ref[...] = jnp.mean(kl1 + kl2, keepdims=True)


def kernel(features, batch_index, W1, b1, gamma, beta, W2, b2,
           W_ih, W_hh, b_ih, b_hh, W3, b3, W4, b4, W5, b5):
    n, d = features.shape
    fdt = jnp.float32
    idx2 = batch_index.astype(jnp.int32).reshape(n, 1)

    bias = 0.0 + 0.0001
    u = jax.random.uniform(jax.random.key(42), (n, 1), fdt)
    eps_g = (bias - (1.0 - bias)) * u + (1.0 - bias)
    noise = jnp.log(eps_g) - jnp.log(1.0 - eps_g)
    u2 = jax.random.uniform(jax.random.key(43), (n, d), fdt)

    ta = _pick_t(n)
    nba = n // ta
    seq = pltpu.CompilerParams(dimension_semantics=("arbitrary",))

    row = lambda bs: pl.BlockSpec(bs, lambda i: (0, 0))
    blk = lambda t, w: pl.BlockSpec((t, w), lambda i: (i, 0))

    cnt0, cnt1 = _sc_counts(idx2.reshape(n))
    cnts = (jnp.sum(cnt0[:, ::16], axis=1) +
            jnp.sum(cnt1[:, ::16], axis=1))[:, None]

    bww = _B + _W
    hsum, hsq, ssum, ssq, _cnt_unused = pl.pallas_call(
        _stats_body,
        grid=(nba,),
        in_specs=[blk(ta, d), blk(ta, 1), row((d, d)), row((1, d))],
        out_specs=[row((1, d)), row((1, d)), row((_B, d)), row((_B, d)),
                   row((_B, 1))],
        out_shape=[jax.ShapeDtypeStruct((1, d), fdt),
                   jax.ShapeDtypeStruct((1, d), fdt),
                   jax.ShapeDtypeStruct((_B, d), fdt),
                   jax.ShapeDtypeStruct((_B, d), fdt),
                   jax.ShapeDtypeStruct((_B, 1), fdt)],
        scratch_shapes=[pltpu.VMEM((bww, d), fdt), pltpu.VMEM((bww, d), fdt),
                        pltpu.VMEM((bww, 1), fdt)],
        compiler_params=seq,
    )(features, idx2, W1.T, b1.reshape(1, d))

    mu_h = hsum / n
    var_h = hsq / n - mu_h * mu_h
    ginv = (gamma.reshape(1, d)) / jnp.sqrt(var_h + 1e-5)
    a_mat = W1.T * ginv
    c_vec = (b1.reshape(1, d) - mu_h) * ginv + beta.reshape(1, d)

    cntc = jnp.maximum(cnts, 1.0)
    mean_seg = ssum / cntc
    var_seg = (ssq - cntc * mean_seg * mean_seg) / jnp.maximum(cntc - 1.0, 1.0)
    std_seg = jnp.sqrt(jnp.maximum(var_seg, 0.0))
    mean_pad = jnp.pad(mean_seg, ((0, _W), (0, 0)))
    std_pad = jnp.pad(std_seg, ((0, _W), (0, 0)))
    bw = _B + _W

    d4 = 4 * d
    bih = b_ih.reshape(1, d4)
    bhh = b_hh.reshape(1, d4)

    lam, pres, r1, s_acc, k_acc = pl.pallas_call(
        _s2s1_body,
        grid=(nba,),
        in_specs=[blk(ta, d), blk(ta, d), blk(ta, 1), blk(ta, 1),
                  row((d, d)), row((1, d)), row((1, d)), row((1, 1)),
                  row((bw, d)), row((bw, d)), row((1, d4)), row((1, d4))],
        out_specs=[blk(ta, 1), row((1, 1)),
                   row((_B, d)), row((_B, 1)), row((_B, d))],
        out_shape=[jax.ShapeDtypeStruct((n, 1), fdt),
                   jax.ShapeDtypeStruct((1, 1), fdt),
                   jax.ShapeDtypeStruct((_B, d), fdt),
                   jax.ShapeDtypeStruct((_B, 1), fdt),
                   jax.ShapeDtypeStruct((_B, d), fdt)],
        scratch_shapes=[pltpu.VMEM((1, d), fdt), pltpu.VMEM((bw, 1), fdt),
                        pltpu.VMEM((bw, 1), fdt), pltpu.VMEM((bw, 1), fdt),
                        pltpu.VMEM((bw, d), fdt), pltpu.VMEM((bw, d), fdt),
                        pltpu.VMEM((bw, 1), fdt), pltpu.VMEM((bw, d), fdt)],
        compiler_params=seq,
    )(features, u2, idx2, noise, a_mat, c_vec, W2.reshape(1, d),
      b2.reshape(1, 1), mean_pad, std_pad, bih, bhh)

    preds, kl = pl.pallas_call(
        _s2s2_body,
        grid=(nba,),
        in_specs=[blk(ta, d), blk(ta, d), blk(ta, 1), blk(ta, 1),
                  row((bw, d)), row((bw, d)), row((_B, d)),
                  row((_B, 1)), row((_B, d)),
                  row((1, d4)), row((1, d4)),
                  row((d, d4)), row((d, d4)), row((d, d4)),
                  row((d, 2 * d)), row((d, 2 * d)), row((1, 2 * d)),
                  row((2 * d, d)), row((1, d)), row((d, d // 2)),
                  row((1, d // 2))],
        out_specs=[row((_B, d // 2)), row((1, 1))],
        out_shape=[jax.ShapeDtypeStruct((_B, d // 2), fdt),
                   jax.ShapeDtypeStruct((1, 1), fdt)],
        scratch_shapes=[pltpu.VMEM((bw, d), fdt), pltpu.VMEM((bw, 1), fdt),
                        pltpu.VMEM((bw, 1), fdt), pltpu.VMEM((bw, 1), fdt),
                        pltpu.VMEM((bw, d), fdt), pltpu.VMEM((bw, d), fdt)],
        compiler_params=seq,
    )(features, u2, idx2, lam, mean_pad, std_pad, r1, s_acc, k_acc,
      bih, bhh, W_ih[:, :d].T, W_ih[:, d:].T, W_hh.T,
      W3[:, :d].T, W3[:, d:].T, b3.reshape(1, 2 * d),
      W4.T, b4.reshape(1, d), W5.T, b5.reshape(1, d // 2))

    return (preds, kl[0, 0], pres[0, 0], lam)


# speedup vs baseline: 1.1091x; 1.1091x over previous
import functools

import jax
import jax.numpy as jnp
from jax import lax
from jax.experimental import pallas as pl
from jax.experimental.pallas import tpu as pltpu
from jax.experimental.pallas import tpu_sc as plsc

_B = 512
_NEG_INF = float("-inf")


def _sc_counts(idx):
    n = idx.shape[0]
    per = (n // 512) * 16
    nlast = n - 31 * per
    mesh = plsc.VectorSubcoreMesh(core_axis_name="c", subcore_axis_name="s")

    @functools.partial(
        pl.kernel, mesh=mesh,
        out_type=[jax.ShapeDtypeStruct((_B, 128), jnp.float32),
                  jax.ShapeDtypeStruct((_B, 128), jnp.float32)],
        scratch_types=[
            pltpu.VMEM((_B, 128), jnp.float32),
            pltpu.VMEM((nlast,), jnp.int32),
            pltpu.VMEM((_B // 128, 128), jnp.int32),
            pltpu.VMEM_SHARED((_B, 128), jnp.float32),
        ],
    )
    def k(idx_hbm, cnt_out0, cnt_out1, cnt, idxb, ident, shcnt):
        c = lax.axis_index("c")
        s = lax.axis_index("s")
        wid = c * 16 + s
        zero16 = jnp.zeros((16,), jnp.float32)
        one16 = jnp.ones((16,), jnp.float32)

        def zb(b, cz):
            for g in range(8):
                cnt[b, pl.ds(g * 16, 16)] = zero16
            return cz
        lax.fori_loop(0, _B, zb, 0)

        ii = lax.iota(jnp.int32, 16)
        for j in range(_B // 128):
            for t in range(8):
                ident[j, pl.ds(t * 16, 16)] = ii + (j * 128 + t * 16)

        @pl.when(s == 0)
        def _():
            pltpu.sync_copy(cnt, shcnt)
        plsc.subcore_barrier()

        pltpu.sync_copy(idx_hbm.at[pl.ds(wid * per, nlast)], idxb)

        def group(g16, cz):
            iv = idxb[pl.ds(g16 * 16, 16)]
            for j in range(16):
                plsc.addupdate(cnt.at[iv[j], pl.ds((j % 8) * 16, 16)], one16)
            return cz
        lax.fori_loop(0, per // 16, group, 0)

        @pl.when(wid == 31)
        def _():
            lax.fori_loop(per // 16, nlast // 16, group, 0)

        plsc.subcore_barrier()
        for j in range(_B // 128):
            pltpu.sync_copy(cnt.at[pl.ds(j * 128, 128)],
                            shcnt.at[ident.at[j]], add=True)
        plsc.subcore_barrier()

        @pl.when((s == 0) & (c == 0))
        def _():
            pltpu.sync_copy(shcnt, cnt_out0)

        @pl.when((s == 0) & (c != 0))
        def _():
            pltpu.sync_copy(shcnt, cnt_out1)

    return k(idx)


def _sig(x):
    return 1.0 / (1.0 + jnp.exp(-x))


def _pick_t(n, candidates=(3200, 1600, 1280, 640, 512, 500, 400, 320, 256,
                           128, 64, 32, 16, 8)):
    for t in candidates:
        if n % t == 0:
            return t
    return n


def _seg_range(idx):
    return jnp.min(idx), jnp.max(idx)


def _stats_body(f_ref, idx_ref, w1t_ref, b1_ref,
                hsum_ref, hsq_ref, ssum_ref, ssq_ref, cnt_ref,
                sacc_ref, qacc_ref, cacc_ref):
    i = pl.program_id(0)
    nsteps = pl.num_programs(0)

    @pl.when(i == 0)
    def _():
        hsum_ref[...] = jnp.zeros_like(hsum_ref)
        hsq_ref[...] = jnp.zeros_like(hsq_ref)
        sacc_ref[...] = jnp.zeros_like(sacc_ref)
        qacc_ref[...] = jnp.zeros_like(qacc_ref)
        cacc_ref[...] = jnp.zeros_like(cacc_ref)

    f = f_ref[...]
    h = jnp.dot(f, w1t_ref[...], preferred_element_type=jnp.float32) + b1_ref[...]
    hsum_ref[...] += jnp.sum(h, axis=0, keepdims=True)
    hsq_ref[...] += jnp.sum(h * h, axis=0, keepdims=True)

    idx = idx_ref[...]
    f2 = f * f
    smin, smax = _seg_range(idx)
    iota_w = lax.broadcasted_iota(jnp.int32, (1, _W), 1)

    def win(w, carry):
        base = smin + w * _W
        oh = ((idx - base) == iota_w).astype(jnp.float32)
        dng = lambda x: lax.dot_general(oh, x, (((0,), (0,)), ((), ())),
                                        preferred_element_type=jnp.float32)
        sacc_ref[pl.ds(base, _W), :] += dng(f)
        qacc_ref[pl.ds(base, _W), :] += dng(f2)
        cacc_ref[pl.ds(base, _W), :] += dng(jnp.ones_like(idx, jnp.float32))
        return carry

    lax.fori_loop(0, (smax - smin) // _W + 1, win, 0)

    @pl.when(i == nsteps - 1)
    def _():
        ssum_ref[...] = sacc_ref[:_B, :]
        ssq_ref[...] = qacc_ref[:_B, :]
        cnt_ref[...] = cacc_ref[:_B, :]


_W = 32


def _seg_softmax_window(oh, e, inw, lam, lneg, f, u2, base,
                        m_ref, den_ref, nums_ref, numf_ref, numu_ref):
    bmax = jnp.max(jnp.where(inw, e, _NEG_INF))
    safe_b = jnp.where(bmax == _NEG_INF, 0.0, bmax)
    exb = jnp.where(inw, jnp.exp(jnp.minimum(e - safe_b, 0.0)), 0.0)
    exn = exb * lneg
    dng = lambda x: lax.dot_general(oh, x, (((0,), (0,)), ((), ())),
                                    preferred_element_type=jnp.float32)
    den_add = dng(exb)
    nums_add = dng(exn)
    numf_add = dng((exb * lam) * f)
    numu_add = dng(exn * u2)
    present = den_add > 0.0
    m_old = m_ref[pl.ds(base, _W), :]
    m_new = jnp.where(present, jnp.maximum(m_old, safe_b), m_old)
    scale = jnp.where(present, jnp.exp(m_old - m_new), 1.0)
    corr = jnp.where(present, jnp.exp(safe_b - m_new), 0.0)
    den_ref[pl.ds(base, _W), :] = den_ref[pl.ds(base, _W), :] * scale + \
        den_add * corr
    nums_ref[pl.ds(base, _W), :] = nums_ref[pl.ds(base, _W), :] * scale + \
        nums_add * corr
    numf_ref[pl.ds(base, _W), :] = numf_ref[pl.ds(base, _W), :] * scale + \
        numf_add * corr
    numu_ref[pl.ds(base, _W), :] = numu_ref[pl.ds(base, _W), :] * scale + \
        numu_add * corr
    m_ref[pl.ds(base, _W), :] = m_new


def _s2s1_body(f_ref, u2_ref, idx_ref, noise_ref, a_ref, c_ref, w2_ref,
               b2_ref, mean_ref, std_ref,
               bih_ref, bhh_ref,
               lam_ref, pres_ref, r1_ref, s_out_ref, k_out_ref,
               q1_ref, m_ref, den_ref, nums_ref, numf_ref, numu_ref,
               sacc_ref, kacc_ref):
    i = pl.program_id(0)
    nsteps = pl.num_programs(0)
    t, d = f_ref.shape

    @pl.when(i == 0)
    def _():
        gates = bih_ref[...] + bhh_ref[...]
        ig = _sig(gates[:, 0:d])
        gg = jnp.tanh(gates[:, 2 * d:3 * d])
        og = _sig(gates[:, 3 * d:4 * d])
        cx1 = ig * gg
        q1_ref[...] = og * jnp.tanh(cx1)
        m_ref[...] = jnp.full_like(m_ref, _NEG_INF)
        den_ref[...] = jnp.zeros_like(den_ref)
        nums_ref[...] = jnp.zeros_like(nums_ref)
        numf_ref[...] = jnp.zeros_like(numf_ref)
        numu_ref[...] = jnp.zeros_like(numu_ref)
        sacc_ref[...] = jnp.zeros_like(sacc_ref)
        kacc_ref[...] = jnp.zeros_like(kacc_ref)
        pres_ref[...] = jnp.zeros_like(pres_ref)

    f = f_ref[...]
    u2 = u2_ref[...]
    hh = jnp.maximum(
        jnp.dot(f, a_ref[...], preferred_element_type=jnp.float32) + c_ref[...],
        0.0)
    p = jnp.sum(hh * w2_ref[...], axis=1, keepdims=True) + b2_ref[...]
    lam = _sig(noise_ref[...] + p)
    lam_ref[...] = lam
    pres_ref[...] += jnp.sum((p > 0.0).astype(jnp.float32), keepdims=True)
    lneg = 1.0 - lam
    lneg2 = lneg * lneg
    idx = idx_ref[...]
    q1 = q1_ref[...]
    dotfq = jnp.sum(f * q1, axis=1, keepdims=True)
    smin, smax = _seg_range(idx)
    iota_w = lax.broadcasted_iota(jnp.int32, (1, _W), 1)

    def win(w, carry):
        base = smin + w * _W
        loc = idx - base
        oh = (loc == iota_w).astype(jnp.float32)
        mu_n = jnp.dot(oh, mean_ref[pl.ds(base, _W), :],
                       preferred_element_type=jnp.float32)
        sg_n = jnp.dot(oh, std_ref[pl.ds(base, _W), :],
                       preferred_element_type=jnp.float32)
        muq = jnp.sum(mu_n * q1, axis=1, keepdims=True)
        u2dot = jnp.sum(u2 * (sg_n * q1), axis=1, keepdims=True)
        e = lam * dotfq + lneg * (muq + u2dot)
        inw = (loc >= 0) & (loc < _W)
        _seg_softmax_window(oh, e, inw, lam, lneg, f, u2, base,
                            m_ref, den_ref, nums_ref, numf_ref, numu_ref)
        dng = lambda x: lax.dot_general(oh, x, (((0,), (0,)), ((), ())),
                                        preferred_element_type=jnp.float32)
        sacc_ref[pl.ds(base, _W), :] += dng(lneg2)
        df = (lam * f + lneg * mu_n) - mu_n
        kacc_ref[pl.ds(base, _W), :] += dng(df * df)
        return carry

    lax.fori_loop(0, (smax - smin) // _W + 1, win, 0)

    @pl.when(i == nsteps - 1)
    def _():
        r1_ref[...] = (numf_ref[:_B, :] + nums_ref[:_B, :] * mean_ref[:_B, :] +
                       numu_ref[:_B, :] * std_ref[:_B, :]) / \
            (den_ref[:_B, :] + 1e-16)
        s_out_ref[...] = sacc_ref[:_B, :]
        k_out_ref[...] = kacc_ref[:_B, :]
        pres_ref[...] = pres_ref[...] * (1.0 / (nsteps * t))


def _s2s2_body(f_ref, u2_ref, idx_ref, lam_ref, mean_ref, std_ref,
               r1_ref, s_in_ref, k_in_ref,
               bih_ref, bhh_ref, wihlt_ref, wihrt_ref, whht_ref,
               w3lt_ref, w3rt_ref, b3_ref, w4t_ref, b4_ref, w5t_ref, b5_ref,
               preds_ref, kl_ref,
               q2_ref, m_ref, den_ref, nums_ref, numf_ref, numu_ref):
    i = pl.program_id(0)
    nsteps = pl.num_programs(0)
    d = f_ref.shape[1]

    @pl.when(i == 0)
    def _():
        gates0 = bih_ref[...] + bhh_ref[...]
        ig0 = _sig(gates0[:, 0:d])
        gg0 = jnp.tanh(gates0[:, 2 * d:3 * d])
        og0 = _sig(gates0[:, 3 * d:4 * d])
        cx1 = ig0 * gg0
        q1 = og0 * jnp.tanh(cx1)
        row = (jnp.dot(q1, wihlt_ref[...], preferred_element_type=jnp.float32)
               + jnp.dot(q1, whht_ref[...], preferred_element_type=jnp.float32)
               + bih_ref[...] + bhh_ref[...])
        gates = jnp.dot(r1_ref[...], wihrt_ref[...],
                        preferred_element_type=jnp.float32) + row
        ig = _sig(gates[:, 0:d])
        fg = _sig(gates[:, d:2 * d])
        gg = jnp.tanh(gates[:, 2 * d:3 * d])
        og = _sig(gates[:, 3 * d:4 * d])
        cx2 = fg * cx1 + ig * gg
        q2_ref[:_B, :] = og * jnp.tanh(cx2)
        q2_ref[_B:, :] = jnp.zeros_like(q2_ref[_B:, :])
        m_ref[...] = jnp.full_like(m_ref, _NEG_INF)
        den_ref[...] = jnp.zeros_like(den_ref)
        nums_ref[...] = jnp.zeros_like(nums_ref)
        numf_ref[...] = jnp.zeros_like(numf_ref)
        numu_ref[...] = jnp.zeros_like(numu_ref)

    f = f_ref[...]
    u2 = u2_ref[...]
    lam = lam_ref[...]
    lneg = 1.0 - lam
    idx = idx_ref[...]
    smin, smax = _seg_range(idx)
    iota_w = lax.broadcasted_iota(jnp.int32, (1, _W), 1)

    def win(w, carry):
        base = smin + w * _W
        loc = idx - base
        oh = (loc == iota_w).astype(jnp.float32)
        mu_n = jnp.dot(oh, mean_ref[pl.ds(base, _W), :],
                       preferred_element_type=jnp.float32)
        sg_n = jnp.dot(oh, std_ref[pl.ds(base, _W), :],
                       preferred_element_type=jnp.float32)
        q_n = jnp.dot(oh, q2_ref[pl.ds(base, _W), :],
                      preferred_element_type=jnp.float32)
        dotfq = jnp.sum(f * q_n, axis=1, keepdims=True)
        muq = jnp.sum(mu_n * q_n, axis=1, keepdims=True)
        u2dot = jnp.sum(u2 * (sg_n * q_n), axis=1, keepdims=True)
        e = lam * dotfq + lneg * (muq + u2dot)
        inw = (loc >= 0) & (loc < _W)
        _seg_softmax_window(oh, e, inw, lam, lneg, f, u2, base,
                            m_ref, den_ref, nums_ref, numf_ref, numu_ref)
        return carry

    lax.fori_loop(0, (smax - smin) // _W + 1, win, 0)

    @pl.when(i == nsteps - 1)
    def _():
        r2 = (numf_ref[:_B, :] + nums_ref[:_B, :] * mean_ref[:_B, :] +
              numu_ref[:_B, :] * std_ref[:_B, :]) / (den_ref[:_B, :] + 1e-16)
        q2 = q2_ref[:_B, :]
        x = jnp.maximum(
            jnp.dot(q2, w3lt_ref[...], preferred_element_type=jnp.float32) +
            jnp.dot(r2, w3rt_ref[...], preferred_element_type=jnp.float32) +
            b3_ref[...], 0.0)
        x = jnp.maximum(
            jnp.dot(x, w4t_ref[...], preferred_element_type=jnp.float32) +
            b4_ref[...], 0.0)
        preds_ref[...] = jnp.dot(x, w5t_ref[...],
                                 preferred_element_type=jnp.float32) + b5_ref[...]
        sg = std_ref[:_B, :]
        sge = (sg + 1e-07) * (sg + 1e-07)
        ms = jnp.mean((sg * sg) / sge, axis=1, keepdims=True)
        kl1 = 0.5 * s_in_ref[...] * ms
        kl2 = k_in_ref[...] / sge
        kl_ref[...] = jnp.mean(kl1 + kl2, keepdims=True)


def kernel(features, batch_index, W1, b1, gamma, beta, W2, b2,
           W_ih, W_hh, b_ih, b_hh, W3, b3, W4, b4, W5, b5):
    n, d = features.shape
    fdt = jnp.float32
    idx2 = batch_index.astype(jnp.int32).reshape(n, 1)

    bias = 0.0 + 0.0001
    u = jax.random.uniform(jax.random.key(42), (n, 1), fdt)
    eps_g = (bias - (1.0 - bias)) * u + (1.0 - bias)
    noise = jnp.log(eps_g) - jnp.log(1.0 - eps_g)
    u2 = jax.random.uniform(jax.random.key(43), (n, d), fdt)

    ta = _pick_t(n)
    nba = n // ta
    seq = pltpu.CompilerParams(dimension_semantics=("arbitrary",))

    row = lambda bs: pl.BlockSpec(bs, lambda i: (0, 0))
    blk = lambda t, w: pl.BlockSpec((t, w), lambda i: (i, 0))

    cnt0, cnt1 = _sc_counts(idx2.reshape(n))
    cnts = (jnp.sum(cnt0[:, ::16], axis=1) +
            jnp.sum(cnt1[:, ::16], axis=1))[:, None]

    bww = _B + _W
    hsum, hsq, ssum, ssq, _cnt_unused = pl.pallas_call(
        _stats_body,
        grid=(nba,),
        in_specs=[blk(ta, d), blk(ta, 1), row((d, d)), row((1, d))],
        out_specs=[row((1, d)), row((1, d)), row((_B, d)), row((_B, d)),
                   row((_B, 1))],
        out_shape=[jax.ShapeDtypeStruct((1, d), fdt),
                   jax.ShapeDtypeStruct((1, d), fdt),
                   jax.ShapeDtypeStruct((_B, d), fdt),
                   jax.ShapeDtypeStruct((_B, d), fdt),
                   jax.ShapeDtypeStruct((_B, 1), fdt)],
        scratch_shapes=[pltpu.VMEM((bww, d), fdt), pltpu.VMEM((bww, d), fdt),
                        pltpu.VMEM((bww, 1), fdt)],
        compiler_params=seq,
    )(features, idx2, W1.T, b1.reshape(1, d))

    mu_h = hsum / n
    var_h = hsq / n - mu_h * mu_h
    ginv = (gamma.reshape(1, d)) / jnp.sqrt(var_h + 1e-5)
    a_mat = W1.T * ginv
    c_vec = (b1.reshape(1, d) - mu_h) * ginv + beta.reshape(1, d)

    cntc = jnp.maximum(cnts, 1.0)
    mean_seg = ssum / cntc
    var_seg = (ssq - cntc * mean_seg * mean_seg) / jnp.maximum(cntc - 1.0, 1.0)
    std_seg = jnp.sqrt(jnp.maximum(var_seg, 0.0))
    mean_pad = jnp.pad(mean_seg, ((0, _W), (0, 0)))
    std_pad = jnp.pad(std_seg, ((0, _W), (0, 0)))
    bw = _B + _W

    d4 = 4 * d
    bih = b_ih.reshape(1, d4)
    bhh = b_hh.reshape(1, d4)

    lam, pres, r1, s_acc, k_acc = pl.pallas_call(
        _s2s1_body,
        grid=(nba,),
        in_specs=[blk(ta, d), blk(ta, d), blk(ta, 1), blk(ta, 1),
                  row((d, d)), row((1, d)), row((1, d)), row((1, 1)),
                  row((bw, d)), row((bw, d)), row((1, d4)), row((1, d4))],
        out_specs=[blk(ta, 1), row((1, 1)),
                   row((_B, d)), row((_B, 1)), row((_B, d))],
        out_shape=[jax.ShapeDtypeStruct((n, 1), fdt),
                   jax.ShapeDtypeStruct((1, 1), fdt),
                   jax.ShapeDtypeStruct((_B, d), fdt),
                   jax.ShapeDtypeStruct((_B, 1), fdt),
                   jax.ShapeDtypeStruct((_B, d), fdt)],
        scratch_shapes=[pltpu.VMEM((1, d), fdt), pltpu.VMEM((bw, 1), fdt),
                        pltpu.VMEM((bw, 1), fdt), pltpu.VMEM((bw, 1), fdt),
                        pltpu.VMEM((bw, d), fdt), pltpu.VMEM((bw, d), fdt),
                        pltpu.VMEM((bw, 1), fdt), pltpu.VMEM((bw, d), fdt)],
        compiler_params=seq,
    )(features, u2, idx2, noise, a_mat, c_vec, W2.reshape(1, d),
      b2.reshape(1, 1), mean_pad, std_pad, bih, bhh)

    preds, kl = pl.pallas_call(
        _s2s2_body,
        grid=(nba,),
        in_specs=[blk(ta, d), blk(ta, d), blk(ta, 1), blk(ta, 1),
                  row((bw, d)), row((bw, d)), row((_B, d)),
                  row((_B, 1)), row((_B, d)),
                  row((1, d4)), row((1, d4)),
                  row((d, d4)), row((d, d4)), row((d, d4)),
                  row((d, 2 * d)), row((d, 2 * d)), row((1, 2 * d)),
                  row((2 * d, d)), row((1, d)), row((d, d // 2)),
                  row((1, d // 2))],
        out_specs=[row((_B, d // 2)), row((1, 1))],
        out_shape=[jax.ShapeDtypeStruct((_B, d // 2), fdt),
                   jax.ShapeDtypeStruct((1, 1), fdt)],
        scratch_shapes=[pltpu.VMEM((bw, d), fdt), pltpu.VMEM((bw, 1), fdt),
                        pltpu.VMEM((bw, 1), fdt), pltpu.VMEM((bw, 1), fdt),
                        pltpu.VMEM((bw, d), fdt), pltpu.VMEM((bw, d), fdt)],
        compiler_params=seq,
    )(features, u2, idx2, lam, mean_pad, std_pad, r1, s_acc, k_acc,
      bih, bhh, W_ih[:, :d].T, W_ih[:, d:].T, W_hh.T,
      W3[:, :d].T, W3[:, d:].T, b3.reshape(1, 2 * d),
      W4.T, b4.reshape(1, d), W5.T, b5.reshape(1, d // 2))

    return (preds, kl[0, 0], pres[0, 0], lam)


# speedup vs baseline: 1.1168x; 1.0070x over previous
import functools

import jax
import jax.numpy as jnp
from jax import lax
from jax.experimental import pallas as pl
from jax.experimental.pallas import tpu as pltpu
from jax.experimental.pallas import tpu_sc as plsc

_B = 512
_NEG_INF = float("-inf")


def _sc_counts(idx):
    n = idx.shape[0]
    per = (n // 512) * 16
    nlast = n - 31 * per
    mesh = plsc.VectorSubcoreMesh(core_axis_name="c", subcore_axis_name="s")

    @functools.partial(
        pl.kernel, mesh=mesh,
        out_type=[jax.ShapeDtypeStruct((_B, 128), jnp.float32),
                  jax.ShapeDtypeStruct((_B, 128), jnp.float32)],
        scratch_types=[
            pltpu.VMEM((_B, 128), jnp.float32),
            pltpu.VMEM((nlast,), jnp.int32),
            pltpu.VMEM((_B // 128, 128), jnp.int32),
            pltpu.VMEM_SHARED((_B, 128), jnp.float32),
        ],
    )
    def k(idx_hbm, cnt_out0, cnt_out1, cnt, idxb, ident, shcnt):
        c = lax.axis_index("c")
        s = lax.axis_index("s")
        wid = c * 16 + s
        zero16 = jnp.zeros((16,), jnp.float32)
        one16 = jnp.ones((16,), jnp.float32)

        def zb(b, cz):
            for g in range(8):
                cnt[b, pl.ds(g * 16, 16)] = zero16
            return cz
        lax.fori_loop(0, _B, zb, 0)

        ii = lax.iota(jnp.int32, 16)
        for j in range(_B // 128):
            for t in range(8):
                ident[j, pl.ds(t * 16, 16)] = ii + (j * 128 + t * 16)

        @pl.when(s == 0)
        def _():
            pltpu.sync_copy(cnt, shcnt)
        plsc.subcore_barrier()

        pltpu.sync_copy(idx_hbm.at[pl.ds(wid * per, nlast)], idxb)

        def group(g16, cz):
            iv = idxb[pl.ds(g16 * 16, 16)]
            for j in range(16):
                plsc.addupdate(cnt.at[iv[j], pl.ds((j % 8) * 16, 16)], one16)
            return cz
        lax.fori_loop(0, per // 16, group, 0)

        @pl.when(wid == 31)
        def _():
            lax.fori_loop(per // 16, nlast // 16, group, 0)

        plsc.subcore_barrier()
        for j in range(_B // 128):
            pltpu.sync_copy(cnt.at[pl.ds(j * 128, 128)],
                            shcnt.at[ident.at[j]], add=True)
        plsc.subcore_barrier()

        @pl.when((s == 0) & (c == 0))
        def _():
            pltpu.sync_copy(shcnt, cnt_out0)

        @pl.when((s == 0) & (c != 0))
        def _():
            pltpu.sync_copy(shcnt, cnt_out1)

    return k(idx)


def _sig(x):
    return 1.0 / (1.0 + jnp.exp(-x))


def _pick_t(n, candidates=(4000, 3200, 1600, 1280, 640, 512, 500, 400, 320,
                           256, 128, 64, 32, 16, 8)):
    for t in candidates:
        if n % t == 0:
            return t
    return n


def _seg_range(idx):
    return jnp.min(idx), jnp.max(idx)


def _stats_body(f_ref, idx_ref, w1t_ref, b1_ref,
                hsum_ref, hsq_ref, ssum_ref, ssq_ref, cnt_ref,
                sacc_ref, qacc_ref, cacc_ref):
    i = pl.program_id(0)
    nsteps = pl.num_programs(0)

    @pl.when(i == 0)
    def _():
        hsum_ref[...] = jnp.zeros_like(hsum_ref)
        hsq_ref[...] = jnp.zeros_like(hsq_ref)
        sacc_ref[...] = jnp.zeros_like(sacc_ref)
        qacc_ref[...] = jnp.zeros_like(qacc_ref)
        cacc_ref[...] = jnp.zeros_like(cacc_ref)

    f = f_ref[...]
    h = jnp.dot(f, w1t_ref[...], preferred_element_type=jnp.float32) + b1_ref[...]
    hsum_ref[...] += jnp.sum(h, axis=0, keepdims=True)
    hsq_ref[...] += jnp.sum(h * h, axis=0, keepdims=True)

    idx = idx_ref[...]
    f2 = f * f
    smin, smax = _seg_range(idx)
    iota_w = lax.broadcasted_iota(jnp.int32, (1, _W), 1)

    def win(w, carry):
        base = smin + w * _W
        oh = ((idx - base) == iota_w).astype(jnp.float32)
        dng = lambda x: lax.dot_general(oh, x, (((0,), (0,)), ((), ())),
                                        preferred_element_type=jnp.float32)
        sacc_ref[pl.ds(base, _W), :] += dng(f)
        qacc_ref[pl.ds(base, _W), :] += dng(f2)
        cacc_ref[pl.ds(base, _W), :] += dng(jnp.ones_like(idx, jnp.float32))
        return carry

    lax.fori_loop(0, (smax - smin) // _W + 1, win, 0)

    @pl.when(i == nsteps - 1)
    def _():
        ssum_ref[...] = sacc_ref[:_B, :]
        ssq_ref[...] = qacc_ref[:_B, :]
        cnt_ref[...] = cacc_ref[:_B, :]


_W = 32


def _seg_softmax_window(oh, e, inw, lam, lneg, f, u2, base,
                        m_ref, den_ref, nums_ref, numf_ref, numu_ref):
    bmax = jnp.max(jnp.where(inw, e, _NEG_INF))
    safe_b = jnp.where(bmax == _NEG_INF, 0.0, bmax)
    exb = jnp.where(inw, jnp.exp(jnp.minimum(e - safe_b, 0.0)), 0.0)
    exn = exb * lneg
    dng = lambda x: lax.dot_general(oh, x, (((0,), (0,)), ((), ())),
                                    preferred_element_type=jnp.float32)
    den_add = dng(exb)
    nums_add = dng(exn)
    numf_add = dng((exb * lam) * f)
    numu_add = dng(exn * u2)
    present = den_add > 0.0
    m_old = m_ref[pl.ds(base, _W), :]
    m_new = jnp.where(present, jnp.maximum(m_old, safe_b), m_old)
    scale = jnp.where(present, jnp.exp(m_old - m_new), 1.0)
    corr = jnp.where(present, jnp.exp(safe_b - m_new), 0.0)
    den_ref[pl.ds(base, _W), :] = den_ref[pl.ds(base, _W), :] * scale + \
        den_add * corr
    nums_ref[pl.ds(base, _W), :] = nums_ref[pl.ds(base, _W), :] * scale + \
        nums_add * corr
    numf_ref[pl.ds(base, _W), :] = numf_ref[pl.ds(base, _W), :] * scale + \
        numf_add * corr
    numu_ref[pl.ds(base, _W), :] = numu_ref[pl.ds(base, _W), :] * scale + \
        numu_add * corr
    m_ref[pl.ds(base, _W), :] = m_new


def _s2s1_body(f_ref, u2_ref, idx_ref, noise_ref, a_ref, c_ref, w2_ref,
               b2_ref, mean_ref, std_ref,
               bih_ref, bhh_ref,
               lam_ref, pres_ref, r1_ref, s_out_ref, k_out_ref,
               q1_ref, m_ref, den_ref, nums_ref, numf_ref, numu_ref,
               sacc_ref, kacc_ref):
    i = pl.program_id(0)
    nsteps = pl.num_programs(0)
    t, d = f_ref.shape

    @pl.when(i == 0)
    def _():
        gates = bih_ref[...] + bhh_ref[...]
        ig = _sig(gates[:, 0:d])
        gg = jnp.tanh(gates[:, 2 * d:3 * d])
        og = _sig(gates[:, 3 * d:4 * d])
        cx1 = ig * gg
        q1_ref[...] = og * jnp.tanh(cx1)
        m_ref[...] = jnp.full_like(m_ref, _NEG_INF)
        den_ref[...] = jnp.zeros_like(den_ref)
        nums_ref[...] = jnp.zeros_like(nums_ref)
        numf_ref[...] = jnp.zeros_like(numf_ref)
        numu_ref[...] = jnp.zeros_like(numu_ref)
        sacc_ref[...] = jnp.zeros_like(sacc_ref)
        kacc_ref[...] = jnp.zeros_like(kacc_ref)
        pres_ref[...] = jnp.zeros_like(pres_ref)

    f = f_ref[...]
    u2 = u2_ref[...]
    hh = jnp.maximum(
        jnp.dot(f, a_ref[...], preferred_element_type=jnp.float32) + c_ref[...],
        0.0)
    p = jnp.sum(hh * w2_ref[...], axis=1, keepdims=True) + b2_ref[...]
    lam = _sig(noise_ref[...] + p)
    lam_ref[...] = lam
    pres_ref[...] += jnp.sum((p > 0.0).astype(jnp.float32), keepdims=True)
    lneg = 1.0 - lam
    lneg2 = lneg * lneg
    idx = idx_ref[...]
    q1 = q1_ref[...]
    dotfq = jnp.sum(f * q1, axis=1, keepdims=True)
    smin, smax = _seg_range(idx)
    iota_w = lax.broadcasted_iota(jnp.int32, (1, _W), 1)

    def win(w, carry):
        base = smin + w * _W
        loc = idx - base
        oh = (loc == iota_w).astype(jnp.float32)
        mu_n = jnp.dot(oh, mean_ref[pl.ds(base, _W), :],
                       preferred_element_type=jnp.float32)
        sg_n = jnp.dot(oh, std_ref[pl.ds(base, _W), :],
                       preferred_element_type=jnp.float32)
        muq = jnp.sum(mu_n * q1, axis=1, keepdims=True)
        u2dot = jnp.sum(u2 * (sg_n * q1), axis=1, keepdims=True)
        e = lam * dotfq + lneg * (muq + u2dot)
        inw = (loc >= 0) & (loc < _W)
        _seg_softmax_window(oh, e, inw, lam, lneg, f, u2, base,
                            m_ref, den_ref, nums_ref, numf_ref, numu_ref)
        dng = lambda x: lax.dot_general(oh, x, (((0,), (0,)), ((), ())),
                                        preferred_element_type=jnp.float32)
        sacc_ref[pl.ds(base, _W), :] += dng(lneg2)
        df = (lam * f + lneg * mu_n) - mu_n
        kacc_ref[pl.ds(base, _W), :] += dng(df * df)
        return carry

    lax.fori_loop(0, (smax - smin) // _W + 1, win, 0)

    @pl.when(i == nsteps - 1)
    def _():
        r1_ref[...] = (numf_ref[:_B, :] + nums_ref[:_B, :] * mean_ref[:_B, :] +
                       numu_ref[:_B, :] * std_ref[:_B, :]) / \
            (den_ref[:_B, :] + 1e-16)
        s_out_ref[...] = sacc_ref[:_B, :]
        k_out_ref[...] = kacc_ref[:_B, :]
        pres_ref[...] = pres_ref[...] * (1.0 / (nsteps * t))


def _s2s2_body(f_ref, u2_ref, idx_ref, lam_ref, mean_ref, std_ref,
               r1_ref, s_in_ref, k_in_ref,
               bih_ref, bhh_ref, wihlt_ref, wihrt_ref, whht_ref,
               w3lt_ref, w3rt_ref, b3_ref, w4t_ref, b4_ref, w5t_ref, b5_ref,
               preds_ref, kl_ref,
               q2_ref, m_ref, den_ref, nums_ref, numf_ref, numu_ref):
    i = pl.program_id(0)
    nsteps = pl.num_programs(0)
    d = f_ref.shape[1]

    @pl.when(i == 0)
    def _():
        gates0 = bih_ref[...] + bhh_ref[...]
        ig0 = _sig(gates0[:, 0:d])
        gg0 = jnp.tanh(gates0[:, 2 * d:3 * d])
        og0 = _sig(gates0[:, 3 * d:4 * d])
        cx1 = ig0 * gg0
        q1 = og0 * jnp.tanh(cx1)
        row = (jnp.dot(q1, wihlt_ref[...], preferred_element_type=jnp.float32)
               + jnp.dot(q1, whht_ref[...], preferred_element_type=jnp.float32)
               + bih_ref[...] + bhh_ref[...])
        gates = jnp.dot(r1_ref[...], wihrt_ref[...],
                        preferred_element_type=jnp.float32) + row
        ig = _sig(gates[:, 0:d])
        fg = _sig(gates[:, d:2 * d])
        gg = jnp.tanh(gates[:, 2 * d:3 * d])
        og = _sig(gates[:, 3 * d:4 * d])
        cx2 = fg * cx1 + ig * gg
        q2_ref[:_B, :] = og * jnp.tanh(cx2)
        q2_ref[_B:, :] = jnp.zeros_like(q2_ref[_B:, :])
        m_ref[...] = jnp.full_like(m_ref, _NEG_INF)
        den_ref[...] = jnp.zeros_like(den_ref)
        nums_ref[...] = jnp.zeros_like(nums_ref)
        numf_ref[...] = jnp.zeros_like(numf_ref)
        numu_ref[...] = jnp.zeros_like(numu_ref)

    f = f_ref[...]
    u2 = u2_ref[...]
    lam = lam_ref[...]
    lneg = 1.0 - lam
    idx = idx_ref[...]
    smin, smax = _seg_range(idx)
    iota_w = lax.broadcasted_iota(jnp.int32, (1, _W), 1)

    def win(w, carry):
        base = smin + w * _W
        loc = idx - base
        oh = (loc == iota_w).astype(jnp.float32)
        mu_n = jnp.dot(oh, mean_ref[pl.ds(base, _W), :],
                       preferred_element_type=jnp.float32)
        sg_n = jnp.dot(oh, std_ref[pl.ds(base, _W), :],
                       preferred_element_type=jnp.float32)
        q_n = jnp.dot(oh, q2_ref[pl.ds(base, _W), :],
                      preferred_element_type=jnp.float32)
        dotfq = jnp.sum(f * q_n, axis=1, keepdims=True)
        muq = jnp.sum(mu_n * q_n, axis=1, keepdims=True)
        u2dot = jnp.sum(u2 * (sg_n * q_n), axis=1, keepdims=True)
        e = lam * dotfq + lneg * (muq + u2dot)
        inw = (loc >= 0) & (loc < _W)
        _seg_softmax_window(oh, e, inw, lam, lneg, f, u2, base,
                            m_ref, den_ref, nums_ref, numf_ref, numu_ref)
        return carry

    lax.fori_loop(0, (smax - smin) // _W + 1, win, 0)

    @pl.when(i == nsteps - 1)
    def _():
        r2 = (numf_ref[:_B, :] + nums_ref[:_B, :] * mean_ref[:_B, :] +
              numu_ref[:_B, :] * std_ref[:_B, :]) / (den_ref[:_B, :] + 1e-16)
        q2 = q2_ref[:_B, :]
        x = jnp.maximum(
            jnp.dot(q2, w3lt_ref[...], preferred_element_type=jnp.float32) +
            jnp.dot(r2, w3rt_ref[...], preferred_element_type=jnp.float32) +
            b3_ref[...], 0.0)
        x = jnp.maximum(
            jnp.dot(x, w4t_ref[...], preferred_element_type=jnp.float32) +
            b4_ref[...], 0.0)
        preds_ref[...] = jnp.dot(x, w5t_ref[...],
                                 preferred_element_type=jnp.float32) + b5_ref[...]
        sg = std_ref[:_B, :]
        sge = (sg + 1e-07) * (sg + 1e-07)
        ms = jnp.mean((sg * sg) / sge, axis=1, keepdims=True)
        kl1 = 0.5 * s_in_ref[...] * ms
        kl2 = k_in_ref[...] / sge
        kl_ref[...] = jnp.mean(kl1 + kl2, keepdims=True)


def kernel(features, batch_index, W1, b1, gamma, beta, W2, b2,
           W_ih, W_hh, b_ih, b_hh, W3, b3, W4, b4, W5, b5):
    n, d = features.shape
    fdt = jnp.float32
    idx2 = batch_index.astype(jnp.int32).reshape(n, 1)

    bias = 0.0 + 0.0001
    u = jax.random.uniform(jax.random.key(42), (n, 1), fdt)
    eps_g = (bias - (1.0 - bias)) * u + (1.0 - bias)
    noise = jnp.log(eps_g) - jnp.log(1.0 - eps_g)
    u2 = jax.random.uniform(jax.random.key(43), (n, d), fdt)

    ta = _pick_t(n)
    nba = n // ta
    seq = pltpu.CompilerParams(dimension_semantics=("arbitrary",))

    row = lambda bs: pl.BlockSpec(bs, lambda i: (0, 0))
    blk = lambda t, w: pl.BlockSpec((t, w), lambda i: (i, 0))

    cnt0, cnt1 = _sc_counts(idx2.reshape(n))
    cnts = (jnp.sum(cnt0[:, ::16], axis=1) +
            jnp.sum(cnt1[:, ::16], axis=1))[:, None]

    bww = _B + _W
    hsum, hsq, ssum, ssq, _cnt_unused = pl.pallas_call(
        _stats_body,
        grid=(nba,),
        in_specs=[blk(ta, d), blk(ta, 1), row((d, d)), row((1, d))],
        out_specs=[row((1, d)), row((1, d)), row((_B, d)), row((_B, d)),
                   row((_B, 1))],
        out_shape=[jax.ShapeDtypeStruct((1, d), fdt),
                   jax.ShapeDtypeStruct((1, d), fdt),
                   jax.ShapeDtypeStruct((_B, d), fdt),
                   jax.ShapeDtypeStruct((_B, d), fdt),
                   jax.ShapeDtypeStruct((_B, 1), fdt)],
        scratch_shapes=[pltpu.VMEM((bww, d), fdt), pltpu.VMEM((bww, d), fdt),
                        pltpu.VMEM((bww, 1), fdt)],
        compiler_params=seq,
    )(features, idx2, W1.T, b1.reshape(1, d))

    mu_h = hsum / n
    var_h = hsq / n - mu_h * mu_h
    ginv = (gamma.reshape(1, d)) / jnp.sqrt(var_h + 1e-5)
    a_mat = W1.T * ginv
    c_vec = (b1.reshape(1, d) - mu_h) * ginv + beta.reshape(1, d)

    cntc = jnp.maximum(cnts, 1.0)
    mean_seg = ssum / cntc
    var_seg = (ssq - cntc * mean_seg * mean_seg) / jnp.maximum(cntc - 1.0, 1.0)
    std_seg = jnp.sqrt(jnp.maximum(var_seg, 0.0))
    mean_pad = jnp.pad(mean_seg, ((0, _W), (0, 0)))
    std_pad = jnp.pad(std_seg, ((0, _W), (0, 0)))
    bw = _B + _W

    d4 = 4 * d
    bih = b_ih.reshape(1, d4)
    bhh = b_hh.reshape(1, d4)

    lam, pres, r1, s_acc, k_acc = pl.pallas_call(
        _s2s1_body,
        grid=(nba,),
        in_specs=[blk(ta, d), blk(ta, d), blk(ta, 1), blk(ta, 1),
                  row((d, d)), row((1, d)), row((1, d)), row((1, 1)),
                  row((bw, d)), row((bw, d)), row((1, d4)), row((1, d4))],
        out_specs=[blk(ta, 1), row((1, 1)),
                   row((_B, d)), row((_B, 1)), row((_B, d))],
        out_shape=[jax.ShapeDtypeStruct((n, 1), fdt),
                   jax.ShapeDtypeStruct((1, 1), fdt),
                   jax.ShapeDtypeStruct((_B, d), fdt),
                   jax.ShapeDtypeStruct((_B, 1), fdt),
                   jax.ShapeDtypeStruct((_B, d), fdt)],
        scratch_shapes=[pltpu.VMEM((1, d), fdt), pltpu.VMEM((bw, 1), fdt),
                        pltpu.VMEM((bw, 1), fdt), pltpu.VMEM((bw, 1), fdt),
                        pltpu.VMEM((bw, d), fdt), pltpu.VMEM((bw, d), fdt),
                        pltpu.VMEM((bw, 1), fdt), pltpu.VMEM((bw, d), fdt)],
        compiler_params=seq,
    )(features, u2, idx2, noise, a_mat, c_vec, W2.reshape(1, d),
      b2.reshape(1, 1), mean_pad, std_pad, bih, bhh)

    preds, kl = pl.pallas_call(
        _s2s2_body,
        grid=(nba,),
        in_specs=[blk(ta, d), blk(ta, d), blk(ta, 1), blk(ta, 1),
                  row((bw, d)), row((bw, d)), row((_B, d)),
                  row((_B, 1)), row((_B, d)),
                  row((1, d4)), row((1, d4)),
                  row((d, d4)), row((d, d4)), row((d, d4)),
                  row((d, 2 * d)), row((d, 2 * d)), row((1, 2 * d)),
                  row((2 * d, d)), row((1, d)), row((d, d // 2)),
                  row((1, d // 2))],
        out_specs=[row((_B, d // 2)), row((1, 1))],
        out_shape=[jax.ShapeDtypeStruct((_B, d // 2), fdt),
                   jax.ShapeDtypeStruct((1, 1), fdt)],
        scratch_shapes=[pltpu.VMEM((bw, d), fdt), pltpu.VMEM((bw, 1), fdt),
                        pltpu.VMEM((bw, 1), fdt), pltpu.VMEM((bw, 1), fdt),
                        pltpu.VMEM((bw, d), fdt), pltpu.VMEM((bw, d), fdt)],
        compiler_params=seq,
    )(features, u2, idx2, lam, mean_pad, std_pad, r1, s_acc, k_acc,
      bih, bhh, W_ih[:, :d].T, W_ih[:, d:].T, W_hh.T,
      W3[:, :d].T, W3[:, d:].T, b3.reshape(1, 2 * d),
      W4.T, b4.reshape(1, d), W5.T, b5.reshape(1, d // 2))

    return (preds, kl[0, 0], pres[0, 0], lam)
